# baseline (device time: 11002 ns/iter reference)
import jax
import jax.numpy as jnp
from jax import lax
from jax.experimental import pallas as pl
from jax.experimental.pallas import tpu as pltpu

N_DEV = 16


def kernel(x):
    m_per, n = x.shape

    def body(x_hbm, out_ref, x_vmem, send_ref, recv_ref, copy_sem,
             send_sems, recv_sems):
        my_pos = lax.axis_index("i")

        barrier_sem = pltpu.get_barrier_semaphore()
        for j in range(N_DEV - 1):
            peer = lax.rem(my_pos + 1 + j, N_DEV)
            pl.semaphore_signal(
                barrier_sem, inc=1,
                device_id=(peer,), device_id_type=pl.DeviceIdType.MESH,
            )

        cp = pltpu.make_async_copy(x_hbm, x_vmem, copy_sem)
        cp.start()
        cp.wait()

        xv = x_vmem[:, :]
        val = jnp.max(xv, axis=0)
        rows = lax.broadcasted_iota(jnp.int32, (m_per, n), 0)
        idx_local = jnp.min(
            jnp.where(xv == val[None, :], rows, jnp.int32(m_per)), axis=0
        )
        gidx = (idx_local + my_pos * m_per).astype(jnp.float32)
        send_ref[0, :] = val
        send_ref[1, :] = gidx

        pl.semaphore_wait(barrier_sem, N_DEV - 1)

        rdmas = []
        for j in range(N_DEV - 1):
            target = lax.rem(my_pos + 2 * N_DEV - 1 - j, N_DEV)
            rdma = pltpu.make_async_remote_copy(
                src_ref=send_ref,
                dst_ref=recv_ref.at[j],
                send_sem=send_sems.at[j],
                recv_sem=recv_sems.at[j],
                device_id=(target,),
                device_id_type=pl.DeviceIdType.MESH,
            )
            rdma.start()
            rdmas.append(rdma)

        acc_val = val
        acc_idx = gidx
        for j in range(N_DEV - 1):
            rdmas[j].wait_recv()
            rv = recv_ref[j, 0, :]
            ri = recv_ref[j, 1, :]
            take = (rv > acc_val) | ((rv == acc_val) & (ri < acc_idx))
            acc_val = jnp.where(take, rv, acc_val)
            acc_idx = jnp.where(take, ri, acc_idx)

        out_ref[0, :] = acc_val
        out_ref[1, :] = acc_idx

        for j in range(N_DEV - 1):
            rdmas[j].wait_send()

    return pl.pallas_call(
        body,
        out_shape=jax.ShapeDtypeStruct((2, n), jnp.float32),
        in_specs=[pl.BlockSpec(memory_space=pl.ANY)],
        out_specs=pl.BlockSpec(memory_space=pltpu.VMEM),
        scratch_shapes=[
            pltpu.VMEM((m_per, n), jnp.float32),
            pltpu.VMEM((2, n), jnp.float32),
            pltpu.VMEM((N_DEV - 1, 2, n), jnp.float32),
            pltpu.SemaphoreType.DMA,
            pltpu.SemaphoreType.DMA((N_DEV - 1,)),
            pltpu.SemaphoreType.DMA((N_DEV - 1,)),
        ],
        compiler_params=pltpu.CompilerParams(collective_id=0),
    )(x)


# device time: 10988 ns/iter; 1.0013x vs baseline; 1.0013x over previous
import jax
import jax.numpy as jnp
from jax import lax
from jax.experimental import pallas as pl
from jax.experimental.pallas import tpu as pltpu

N_DEV = 16


def kernel(x):
    m_per, n = x.shape

    def body(x_hbm, out_ref, x_vmem, send_ref, recv_ref, copy_sem,
             send_sems, recv_sems):
        my_pos = lax.axis_index("i")

        barrier_sem = pltpu.get_barrier_semaphore()
        for j in range(N_DEV - 1):
            peer = lax.rem(my_pos + 1 + j, N_DEV)
            pl.semaphore_signal(
                barrier_sem, inc=1,
                device_id=(peer,), device_id_type=pl.DeviceIdType.MESH,
            )

        n_chunks = 4
        rows_per = m_per // n_chunks
        cps = [
            pltpu.make_async_copy(
                x_hbm.at[pl.ds(c * rows_per, rows_per), :],
                x_vmem.at[c],
                copy_sem.at[c],
            )
            for c in range(n_chunks)
        ]
        for cp in cps:
            cp.start()

        rows = lax.broadcasted_iota(jnp.int32, (rows_per, n), 0)
        val = None
        for c in range(n_chunks):
            cps[c].wait()
            xv = x_vmem[c]
            cval = jnp.max(xv, axis=0)
            cidx = jnp.min(
                jnp.where(xv == cval[None, :], rows, jnp.int32(rows_per)),
                axis=0,
            ) + jnp.int32(c * rows_per)
            if val is None:
                val, idx_local = cval, cidx
            else:
                take = (cval > val) | ((cval == val) & (cidx < idx_local))
                val = jnp.where(take, cval, val)
                idx_local = jnp.where(take, cidx, idx_local)
        gidx = (idx_local + my_pos * m_per).astype(jnp.float32)
        send_ref[0, :] = val
        send_ref[1, :] = gidx

        pl.semaphore_wait(barrier_sem, N_DEV - 1)

        rdmas = []
        for j in range(N_DEV - 1):
            target = lax.rem(my_pos + 2 * N_DEV - 1 - j, N_DEV)
            rdma = pltpu.make_async_remote_copy(
                src_ref=send_ref,
                dst_ref=recv_ref.at[j],
                send_sem=send_sems.at[j],
                recv_sem=recv_sems.at[j],
                device_id=(target,),
                device_id_type=pl.DeviceIdType.MESH,
            )
            rdma.start()
            rdmas.append(rdma)

        acc_val = val
        acc_idx = gidx
        for j in range(N_DEV - 1):
            rdmas[j].wait_recv()
            rv = recv_ref[j, 0, :]
            ri = recv_ref[j, 1, :]
            take = (rv > acc_val) | ((rv == acc_val) & (ri < acc_idx))
            acc_val = jnp.where(take, rv, acc_val)
            acc_idx = jnp.where(take, ri, acc_idx)

        out_ref[0, :] = acc_val
        out_ref[1, :] = acc_idx

        for j in range(N_DEV - 1):
            rdmas[j].wait_send()

    return pl.pallas_call(
        body,
        out_shape=jax.ShapeDtypeStruct((2, n), jnp.float32),
        in_specs=[pl.BlockSpec(memory_space=pl.ANY)],
        out_specs=pl.BlockSpec(memory_space=pltpu.VMEM),
        scratch_shapes=[
            pltpu.VMEM((4, m_per // 4, n), jnp.float32),
            pltpu.VMEM((2, n), jnp.float32),
            pltpu.VMEM((N_DEV - 1, 2, n), jnp.float32),
            pltpu.SemaphoreType.DMA((4,)),
            pltpu.SemaphoreType.DMA((N_DEV - 1,)),
            pltpu.SemaphoreType.DMA((N_DEV - 1,)),
        ],
        compiler_params=pltpu.CompilerParams(collective_id=0),
    )(x)


# device time: 10903 ns/iter; 1.0091x vs baseline; 1.0078x over previous
import jax
import jax.numpy as jnp
from jax import lax
from jax.experimental import pallas as pl
from jax.experimental.pallas import tpu as pltpu

N_DEV = 16


def kernel(x):
    m_per, n = x.shape

    def body(x_ref, out_ref, send_ref, recv_ref, send_sems, recv_sems):
        my_pos = lax.axis_index("i")

        barrier_sem = pltpu.get_barrier_semaphore()
        for j in range(N_DEV - 1):
            peer = lax.rem(my_pos + 1 + j, N_DEV)
            pl.semaphore_signal(
                barrier_sem, inc=1,
                device_id=(peer,), device_id_type=pl.DeviceIdType.MESH,
            )

        xv = x_ref[:, :]
        val = jnp.max(xv, axis=0)
        rows = lax.broadcasted_iota(jnp.int32, (m_per, n), 0)
        idx_local = jnp.min(
            jnp.where(xv == val[None, :], rows, jnp.int32(m_per)), axis=0
        )
        gidx = (idx_local + my_pos * m_per).astype(jnp.float32)
        send_ref[0, :] = val
        send_ref[1, :] = gidx

        pl.semaphore_wait(barrier_sem, N_DEV - 1)

        rdmas = []
        for j in range(N_DEV - 1):
            target = lax.rem(my_pos + 2 * N_DEV - 1 - j, N_DEV)
            rdma = pltpu.make_async_remote_copy(
                src_ref=send_ref,
                dst_ref=recv_ref.at[j],
                send_sem=send_sems.at[j],
                recv_sem=recv_sems.at[j],
                device_id=(target,),
                device_id_type=pl.DeviceIdType.MESH,
            )
            rdma.start()
            rdmas.append(rdma)

        acc_val = val
        acc_idx = gidx
        for j in range(N_DEV - 1):
            rdmas[j].wait_recv()
            rv = recv_ref[j, 0, :]
            ri = recv_ref[j, 1, :]
            take = (rv > acc_val) | ((rv == acc_val) & (ri < acc_idx))
            acc_val = jnp.where(take, rv, acc_val)
            acc_idx = jnp.where(take, ri, acc_idx)

        out_ref[0, :] = acc_val
        out_ref[1, :] = acc_idx

        for j in range(N_DEV - 1):
            rdmas[j].wait_send()

    return pl.pallas_call(
        body,
        out_shape=jax.ShapeDtypeStruct((2, n), jnp.float32),
        in_specs=[pl.BlockSpec(memory_space=pltpu.VMEM)],
        out_specs=pl.BlockSpec(memory_space=pltpu.VMEM),
        scratch_shapes=[
            pltpu.VMEM((2, n), jnp.float32),
            pltpu.VMEM((N_DEV - 1, 2, n), jnp.float32),
            pltpu.SemaphoreType.DMA((N_DEV - 1,)),
            pltpu.SemaphoreType.DMA((N_DEV - 1,)),
        ],
        compiler_params=pltpu.CompilerParams(collective_id=0),
    )(x)
